# baseline (speedup 1.0000x reference)
import functools

import jax
import jax.numpy as jnp
from jax import lax
from jax.experimental import pallas as pl
from jax.experimental.pallas import tpu as pltpu
from jax.experimental.pallas import tpu_sc as plsc

NN = 10000
NP = 10240
E = 320000
DE = 16
H = 128
NG = 16

NW = 32

NCHUNK = 5
ECH = E // NCHUNK
EPWC = ECH // NW

CH = 80
NCHC = EPWC // CH
NB = 5
NGRPC = NCHC // NB

CHS = 40
NCHSC = EPWC // CHS
NBS = 2
NGRPSC = NCHSC // NBS

EPW = E // NW
NCH = EPW // CH
RPT = NP // 16

NBL = 1024
EB = 2000

_MESH_KW = dict(core_axis_name="c", subcore_axis_name="s")


def _sc_gather(xw, rowc):
    mesh = plsc.VectorSubcoreMesh(**_MESH_KW)

    @functools.partial(
        pl.kernel,
        out_type=jax.ShapeDtypeStruct((ECH, H), jnp.float32),
        mesh=mesh,
        scratch_types=[
            pltpu.VMEM((NCHC, CH), jnp.int32),
            pltpu.VMEM((NB, CH, H), jnp.float32),
            pltpu.SemaphoreType.DMA,
            pltpu.SemaphoreType.DMA,
        ],
    )
    def k(xw_hbm, row_hbm, out_hbm, idx_v, buf_v, gsem, ssem):
        wid = lax.axis_index("c") * 16 + lax.axis_index("s")
        base = wid * EPWC
        pltpu.sync_copy(row_hbm.at[wid], idx_v)

        @pl.loop(0, NGRPC)
        def _grp(g):
            j0 = g * NB
            for b in range(NB):
                pltpu.async_copy(xw_hbm.at[idx_v.at[j0 + b]], buf_v.at[b], gsem)
            for b in range(NB):
                pltpu.make_async_copy(
                    xw_hbm.at[idx_v.at[j0 + b]], buf_v.at[b], gsem).wait()
            for b in range(NB):
                pltpu.async_copy(
                    buf_v.at[b], out_hbm.at[pl.ds(base + (j0 + b) * CH, CH)], ssem)
            for b in range(NB):
                pltpu.make_async_copy(
                    buf_v.at[b], out_hbm.at[pl.ds(base + (j0 + b) * CH, CH)], ssem).wait()

    return k(xw, rowc)


def _sc_scatter(m, colcs, zrows):
    mesh = plsc.VectorSubcoreMesh(**_MESH_KW)

    @functools.partial(
        pl.kernel,
        out_type=jax.ShapeDtypeStruct((2, NP, H), jnp.float32),
        mesh=mesh,
        scratch_types=[
            pltpu.VMEM((NCHSC, CHS), jnp.int32),
            pltpu.VMEM((NBS, CHS, H), jnp.float32),
            pltpu.VMEM_SHARED((NP, H), jnp.float32),
            pltpu.SemaphoreType.DMA,
        ],
    )
    def k(m_hbm, col_hbm, z_hbm, out_hbm, idx_v, buf_v, acc_sh, lsem):
        cid = lax.axis_index("c")
        sid = lax.axis_index("s")
        wid = cid * 16 + sid
        base = wid * EPWC
        pltpu.sync_copy(z_hbm, acc_sh.at[pl.ds(sid * RPT, RPT)])
        pltpu.sync_copy(col_hbm.at[wid], idx_v)
        plsc.subcore_barrier()

        def _issue(g):
            for b in range(NBS):
                pltpu.async_copy(
                    m_hbm.at[pl.ds(base + (g * NBS + b) * CHS, CHS)],
                    buf_v.at[b], lsem)

        _issue(0)

        @pl.loop(0, NGRPSC)
        def _grp(g):
            for b in range(NBS):
                pltpu.make_async_copy(
                    m_hbm.at[pl.ds(base + (g * NBS + b) * CHS, CHS)],
                    buf_v.at[b], lsem).wait()
            for b in range(NBS):
                pltpu.sync_copy(buf_v.at[b], acc_sh.at[idx_v.at[g * NBS + b]], add=True)

            @pl.when(g + 1 < NGRPSC)
            def _():
                _issue(g + 1)

        plsc.subcore_barrier()
        pltpu.sync_copy(acc_sh.at[pl.ds(sid * RPT, RPT)],
                        out_hbm.at[cid, pl.ds(sid * RPT, RPT)])

    return k(m, colcs, zrows)


def _sc_scatter2(ma, mb, col2, zrows):
    mesh = plsc.VectorSubcoreMesh(**_MESH_KW)
    EPS2 = ECH // 16
    NCHS2 = EPS2 // CHS
    NGRPS2 = NCHS2 // NBS

    @functools.partial(
        pl.kernel,
        out_type=jax.ShapeDtypeStruct((2, NP, H), jnp.float32),
        mesh=mesh,
        scratch_types=[
            pltpu.VMEM((NCHS2, CHS), jnp.int32),
            pltpu.VMEM((NBS, CHS, H), jnp.float32),
            pltpu.VMEM_SHARED((NP, H), jnp.float32),
            pltpu.SemaphoreType.DMA,
        ],
    )
    def k(ma_hbm, mb_hbm, col_hbm, z_hbm, out_hbm, idx_v, buf_v, acc_sh, lsem):
        cid = lax.axis_index("c")
        sid = lax.axis_index("s")
        base = sid * EPS2
        pltpu.sync_copy(z_hbm, acc_sh.at[pl.ds(sid * RPT, RPT)])
        pltpu.sync_copy(col_hbm.at[cid, sid], idx_v)
        plsc.subcore_barrier()

        def run(m_hbm):
            def _issue(g):
                for b in range(NBS):
                    pltpu.async_copy(
                        m_hbm.at[pl.ds(base + (g * NBS + b) * CHS, CHS)],
                        buf_v.at[b], lsem)

            _issue(0)

            @pl.loop(0, NGRPS2)
            def _grp(g):
                for b in range(NBS):
                    pltpu.make_async_copy(
                        m_hbm.at[pl.ds(base + (g * NBS + b) * CHS, CHS)],
                        buf_v.at[b], lsem).wait()
                for b in range(NBS):
                    pltpu.sync_copy(buf_v.at[b],
                                    acc_sh.at[idx_v.at[g * NBS + b]], add=True)

                @pl.when(g + 1 < NGRPS2)
                def _():
                    _issue(g + 1)

        @pl.when(cid == 0)
        def _():
            run(ma_hbm)

        @pl.when(cid == 1)
        def _():
            run(mb_hbm)

        plsc.subcore_barrier()
        pltpu.sync_copy(acc_sh.at[pl.ds(sid * RPT, RPT)],
                        out_hbm.at[cid, pl.ds(sid * RPT, RPT)])

    return k(ma, mb, col2, zrows)


def _sc_counts(col3, zrows, ones_rows):
    mesh = plsc.VectorSubcoreMesh(**_MESH_KW)

    @functools.partial(
        pl.kernel,
        out_type=jax.ShapeDtypeStruct((2, NP, H), jnp.float32),
        mesh=mesh,
        scratch_types=[
            pltpu.VMEM((NCH, CH), jnp.int32),
            pltpu.VMEM((CH, H), jnp.float32),
            pltpu.VMEM_SHARED((NP, H), jnp.float32),
        ],
    )
    def k(col_hbm, z_hbm, ones_hbm, out_hbm, idx_v, ones_v, acc_sh):
        cid = lax.axis_index("c")
        sid = lax.axis_index("s")
        wid = cid * 16 + sid
        pltpu.sync_copy(z_hbm, acc_sh.at[pl.ds(sid * RPT, RPT)])
        pltpu.sync_copy(ones_hbm, ones_v)
        pltpu.sync_copy(col_hbm.at[wid], idx_v)
        plsc.subcore_barrier()

        @pl.loop(0, NCH)
        def _j(j):
            pltpu.sync_copy(ones_v, acc_sh.at[idx_v.at[j]], add=True)

        plsc.subcore_barrier()
        pltpu.sync_copy(acc_sh.at[pl.ds(sid * RPT, RPT)],
                        out_hbm.at[cid, pl.ds(sid * RPT, RPT)])

    return k(col3, zrows, ones_rows)


def _ln_in(x, g, b):
    mu = jnp.mean(x, axis=-1, keepdims=True)
    var = jnp.mean((x - mu) ** 2, axis=-1, keepdims=True)
    return (x - mu) * lax.rsqrt(var + 1e-5) * g + b


def _gelu_in(x):
    return 0.5 * x * (1.0 + lax.erf(x * 0.7071067811865476))


def _tc_node_encoder(nf, nW, nb, ng, nbe, w1t, b1):
    def body(nf_r, nW_r, nb_r, ng_r, nbe_r, w1t_r, b1_r, x_r, xw_r):
        t = jnp.dot(nf_r[...], nW_r[...], preferred_element_type=jnp.float32)
        x = _gelu_in(_ln_in(t + nb_r[...], ng_r[...], nbe_r[...]))
        x_r[...] = x
        xw_r[...] = jnp.dot(x, w1t_r[...],
                            preferred_element_type=jnp.float32) + b1_r[...]

    vec = pl.BlockSpec((1, H), lambda i: (0, 0))
    mat = pl.BlockSpec((H, H), lambda i: (0, 0))
    blk = pl.BlockSpec((NBL, H), lambda i: (i, 0))
    return pl.pallas_call(
        body,
        grid=(NP // NBL,),
        in_specs=[blk, mat, vec, vec, vec, mat, vec],
        out_specs=[blk, blk],
        out_shape=[jax.ShapeDtypeStruct((NP, H), jnp.float32)] * 2,
        compiler_params=pltpu.CompilerParams(
            dimension_semantics=("arbitrary",)),
    )(nf, nW, nb, ng, nbe, w1t, b1)


def _tc_messages(ef, gx, eW, eb, eg, ebe, w1b, g1, be1, w2, b2, g2, be2):
    def body(ef_r, gx_r, eW_r, eb_r, eg_r, ebe_r, w1b_r, g1_r, be1_r,
             w2_r, b2_r, g2_r, be2_r, m_r):
        ea = jnp.dot(ef_r[...], eW_r[...], preferred_element_type=jnp.float32)
        ea = _gelu_in(_ln_in(ea + eb_r[...], eg_r[...], ebe_r[...]))
        pre = gx_r[...] + jnp.dot(ea, w1b_r[...],
                                  preferred_element_type=jnp.float32)
        h = _gelu_in(_ln_in(pre, g1_r[...], be1_r[...]))
        m = jnp.dot(h, w2_r[...], preferred_element_type=jnp.float32)
        m_r[...] = _gelu_in(_ln_in(m + b2_r[...], g2_r[...], be2_r[...]))

    vec = pl.BlockSpec((1, H), lambda i: (0, 0))
    mat = pl.BlockSpec((H, H), lambda i: (0, 0))
    return pl.pallas_call(
        body,
        grid=(ECH // EB,),
        in_specs=[
            pl.BlockSpec((EB, DE), lambda i: (i, 0)),
            pl.BlockSpec((EB, H), lambda i: (i, 0)),
            pl.BlockSpec((DE, H), lambda i: (0, 0)),
            vec, vec, vec, mat, vec, vec, mat, vec, vec, vec,
        ],
        out_specs=pl.BlockSpec((EB, H), lambda i: (i, 0)),
        out_shape=jax.ShapeDtypeStruct((ECH, H), jnp.float32),
        compiler_params=pltpu.CompilerParams(
            dimension_semantics=("arbitrary",)),
    )(ef, gx, eW, eb, eg, ebe, w1b, g1, be1, w2, b2, g2, be2)


def _agg_from_parts(acc_refs, c_r):
    cnt = c_r[0, :, 0:1] + c_r[1, :, 0:1]
    s = acc_refs[0][0] + acc_refs[0][1]
    for a_r in acc_refs[1:]:
        s = s + a_r[0] + a_r[1]
    return s / jnp.maximum(cnt, 1.0)


def _tc_update(x, parts, cnt2, w1t, b1):
    npart = len(parts)

    def body(*refs):
        x_r = refs[0]
        accs = refs[1:1 + npart]
        c_r, w1t_r, b1_r, xn_r, xw_r = refs[1 + npart:]
        xn = x_r[...] + _agg_from_parts(accs, c_r)
        xn_r[...] = xn
        xw_r[...] = jnp.dot(xn, w1t_r[...],
                            preferred_element_type=jnp.float32) + b1_r[...]

    vec = pl.BlockSpec((1, H), lambda i: (0, 0))
    mat = pl.BlockSpec((H, H), lambda i: (0, 0))
    blk = pl.BlockSpec((NBL, H), lambda i: (i, 0))
    acc = pl.BlockSpec((2, NBL, H), lambda i: (0, i, 0))
    return pl.pallas_call(
        body,
        grid=(NP // NBL,),
        in_specs=[blk] + [acc] * npart + [acc, mat, vec],
        out_specs=[blk, blk],
        out_shape=[jax.ShapeDtypeStruct((NP, H), jnp.float32)] * 2,
        compiler_params=pltpu.CompilerParams(
            dimension_semantics=("arbitrary",)),
    )(x, *parts, cnt2, w1t, b1)


def _tc_final(x, parts, cnt2, bi, gnorm, bnorm):
    nblk = NP // NBL
    npart = len(parts)

    def body(*refs):
        x_r = refs[0]
        accs = refs[1:1 + npart]
        c_r, bi_r, g_r, b_r, out_r, gsum, gcnt = refs[1 + npart:]
        i = pl.program_id(0)

        @pl.when(i == 0)
        def _():
            gsum[...] = jnp.zeros_like(gsum)
            gcnt[...] = jnp.zeros_like(gcnt)

        xn = x_r[...] + _agg_from_parts(accs, c_r)
        xn = _ln_in(xn, g_r[...], b_r[...])
        bib = jnp.broadcast_to(bi_r[...], (NG, NBL))
        iot = lax.broadcasted_iota(jnp.int32, (NG, NBL), 0).astype(jnp.float32)
        oh = jnp.where(iot == bib, 1.0, 0.0)
        gsum[...] += jnp.dot(oh, xn, preferred_element_type=jnp.float32)
        gcnt[...] += jnp.broadcast_to(
            jnp.sum(oh, axis=1, keepdims=True), (NG, H))

        @pl.when(i == nblk - 1)
        def _():
            out_r[...] = gsum[...] / jnp.maximum(gcnt[...], 1.0)

    vec = pl.BlockSpec((1, H), lambda i: (0, 0))
    acc = pl.BlockSpec((2, NBL, H), lambda i: (0, i, 0))
    return pl.pallas_call(
        body,
        grid=(nblk,),
        in_specs=[pl.BlockSpec((NBL, H), lambda i: (i, 0))]
        + [acc] * npart
        + [acc, pl.BlockSpec((1, NBL), lambda i: (0, i)), vec, vec],
        out_specs=pl.BlockSpec((NG, H), lambda i: (0, 0)),
        out_shape=jax.ShapeDtypeStruct((NG, H), jnp.float32),
        scratch_shapes=[pltpu.VMEM((NG, H), jnp.float32)] * 2,
        compiler_params=pltpu.CompilerParams(
            dimension_semantics=("arbitrary",)),
    )(x, *parts, cnt2, bi, gnorm, bnorm)


def kernel(node_feats, edge_feats, edge_index, batch_index, params):
    f32 = jnp.float32
    p = params
    layers = p['layers']

    row4 = edge_index[0].astype(jnp.int32).reshape(NCHUNK, NW, NCHC, CH)
    col_i32 = edge_index[1].astype(jnp.int32)
    col3 = col_i32.reshape(NW, NCH, CH)
    col4s = col_i32.reshape(NCHUNK, NW, NCHSC, CHS)
    colp = col_i32.reshape(NCHUNK, 16, ECH // 16 // CHS, CHS)
    ef4 = edge_feats.reshape(NCHUNK, ECH, DE)
    nf_p = jnp.pad(node_feats, ((0, NP - NN), (0, 0)))
    bi = jnp.pad(batch_index.astype(f32), (0, NP - NN),
                 constant_values=float(NG)).reshape(1, NP)
    z128 = jnp.zeros((RPT, H), f32)
    ones128 = jnp.ones((CH, H), f32)

    def v2d(v):
        return v.reshape(1, H)

    w1t = [lp['W1'][:H] for lp in layers]
    w1b = [lp['W1'][H:] for lp in layers]

    cnt2 = _sc_counts(col3, z128, ones128)

    x, xw = _tc_node_encoder(nf_p, p['node_W'], v2d(p['node_b']),
                             v2d(p['node_g']), v2d(p['node_be']),
                             w1t[0], v2d(layers[0]['b1']))

    parts = None
    for i in range(3):
        lp = layers[i]
        mc = []
        for kc in range(NCHUNK):
            gx = _sc_gather(xw, row4[kc])
            mc.append(_tc_messages(
                ef4[kc], gx, p['edge_W'], v2d(p['edge_b']),
                v2d(p['edge_g']), v2d(p['edge_be']), w1b[i],
                v2d(lp['g1']), v2d(lp['be1']), lp['W2'],
                v2d(lp['b2']), v2d(lp['g2']), v2d(lp['be2'])))
        parts = [
            _sc_scatter2(mc[0], mc[1], colp[0:2], z128),
            _sc_scatter2(mc[2], mc[3], colp[2:4], z128),
            _sc_scatter(mc[4], col4s[4], z128),
        ]
        if i < 2:
            x, xw = _tc_update(x, parts, cnt2, w1t[i + 1],
                               v2d(layers[i + 1]['b1']))

    return _tc_final(x, parts, cnt2, bi, v2d(p['norm_g']), v2d(p['norm_b']))


# speedup vs baseline: 1.0368x; 1.0368x over previous
import functools

import jax
import jax.numpy as jnp
from jax import lax
from jax.experimental import pallas as pl
from jax.experimental.pallas import tpu as pltpu
from jax.experimental.pallas import tpu_sc as plsc

NN = 10000
NP = 10240
E = 320000
DE = 16
H = 128
NG = 16

NW = 32

NCHUNK = 5
ECH = E // NCHUNK
EPWC = ECH // NW

CH = 80
NCHC = EPWC // CH
NB = 5
NGRPC = NCHC // NB

CHS = 40
NCHSC = EPWC // CHS
NBS = 2
NGRPSC = NCHSC // NBS

EPW = E // NW
NCH = EPW // CH
RPT = NP // 16

NBL = 1024
EB = 2000

_MESH_KW = dict(core_axis_name="c", subcore_axis_name="s")


def _sc_gather(xw, rowc):
    mesh = plsc.VectorSubcoreMesh(**_MESH_KW)

    @functools.partial(
        pl.kernel,
        out_type=jax.ShapeDtypeStruct((ECH, H), jnp.float32),
        mesh=mesh,
        scratch_types=[
            pltpu.VMEM((NCHC, CH), jnp.int32),
            pltpu.VMEM((NB, CH, H), jnp.float32),
            pltpu.SemaphoreType.DMA,
            pltpu.SemaphoreType.DMA,
        ],
    )
    def k(xw_hbm, row_hbm, out_hbm, idx_v, buf_v, gsem, ssem):
        wid = lax.axis_index("c") * 16 + lax.axis_index("s")
        base = wid * EPWC
        pltpu.sync_copy(row_hbm.at[wid], idx_v)

        @pl.loop(0, NGRPC)
        def _grp(g):
            j0 = g * NB
            for b in range(NB):
                pltpu.async_copy(xw_hbm.at[idx_v.at[j0 + b]], buf_v.at[b], gsem)
            for b in range(NB):
                pltpu.make_async_copy(
                    xw_hbm.at[idx_v.at[j0 + b]], buf_v.at[b], gsem).wait()
            for b in range(NB):
                pltpu.async_copy(
                    buf_v.at[b], out_hbm.at[pl.ds(base + (j0 + b) * CH, CH)], ssem)
            for b in range(NB):
                pltpu.make_async_copy(
                    buf_v.at[b], out_hbm.at[pl.ds(base + (j0 + b) * CH, CH)], ssem).wait()

    return k(xw, rowc)


def _sc_scatter(m, colcs, zrows):
    mesh = plsc.VectorSubcoreMesh(**_MESH_KW)

    @functools.partial(
        pl.kernel,
        out_type=jax.ShapeDtypeStruct((2, NP, H), jnp.float32),
        mesh=mesh,
        scratch_types=[
            pltpu.VMEM((NCHSC, CHS), jnp.int32),
            pltpu.VMEM((NBS, CHS, H), jnp.float32),
            pltpu.VMEM_SHARED((NP, H), jnp.float32),
            pltpu.SemaphoreType.DMA,
        ],
    )
    def k(m_hbm, col_hbm, z_hbm, out_hbm, idx_v, buf_v, acc_sh, lsem):
        cid = lax.axis_index("c")
        sid = lax.axis_index("s")
        wid = cid * 16 + sid
        base = wid * EPWC
        pltpu.sync_copy(z_hbm, acc_sh.at[pl.ds(sid * RPT, RPT)])
        pltpu.sync_copy(col_hbm.at[wid], idx_v)
        plsc.subcore_barrier()

        def _issue(g):
            for b in range(NBS):
                pltpu.async_copy(
                    m_hbm.at[pl.ds(base + (g * NBS + b) * CHS, CHS)],
                    buf_v.at[b], lsem)

        _issue(0)

        @pl.loop(0, NGRPSC)
        def _grp(g):
            for b in range(NBS):
                pltpu.make_async_copy(
                    m_hbm.at[pl.ds(base + (g * NBS + b) * CHS, CHS)],
                    buf_v.at[b], lsem).wait()
            for b in range(NBS):
                pltpu.sync_copy(buf_v.at[b], acc_sh.at[idx_v.at[g * NBS + b]], add=True)

            @pl.when(g + 1 < NGRPSC)
            def _():
                _issue(g + 1)

        plsc.subcore_barrier()
        pltpu.sync_copy(acc_sh.at[pl.ds(sid * RPT, RPT)],
                        out_hbm.at[cid, pl.ds(sid * RPT, RPT)])

    return k(m, colcs, zrows)


def _sc_counts(col3, zrows, ones_rows):
    mesh = plsc.VectorSubcoreMesh(**_MESH_KW)

    @functools.partial(
        pl.kernel,
        out_type=jax.ShapeDtypeStruct((2, NP, H), jnp.float32),
        mesh=mesh,
        scratch_types=[
            pltpu.VMEM((NCH, CH), jnp.int32),
            pltpu.VMEM((CH, H), jnp.float32),
            pltpu.VMEM_SHARED((NP, H), jnp.float32),
        ],
    )
    def k(col_hbm, z_hbm, ones_hbm, out_hbm, idx_v, ones_v, acc_sh):
        cid = lax.axis_index("c")
        sid = lax.axis_index("s")
        wid = cid * 16 + sid
        pltpu.sync_copy(z_hbm, acc_sh.at[pl.ds(sid * RPT, RPT)])
        pltpu.sync_copy(ones_hbm, ones_v)
        pltpu.sync_copy(col_hbm.at[wid], idx_v)
        plsc.subcore_barrier()

        @pl.loop(0, NCH)
        def _j(j):
            pltpu.sync_copy(ones_v, acc_sh.at[idx_v.at[j]], add=True)

        plsc.subcore_barrier()
        pltpu.sync_copy(acc_sh.at[pl.ds(sid * RPT, RPT)],
                        out_hbm.at[cid, pl.ds(sid * RPT, RPT)])

    return k(col3, zrows, ones_rows)


def _ln_in(x, g, b):
    mu = jnp.mean(x, axis=-1, keepdims=True)
    var = jnp.mean((x - mu) ** 2, axis=-1, keepdims=True)
    return (x - mu) * lax.rsqrt(var + 1e-5) * g + b


def _gelu_in(x):
    return 0.5 * x * (1.0 + lax.erf(x * 0.7071067811865476))


def _tc_node_encoder(nf, nW, nb, ng, nbe, w1t, b1):
    def body(nf_r, nW_r, nb_r, ng_r, nbe_r, w1t_r, b1_r, x_r, xw_r):
        t = jnp.dot(nf_r[...], nW_r[...], preferred_element_type=jnp.float32)
        x = _gelu_in(_ln_in(t + nb_r[...], ng_r[...], nbe_r[...]))
        x_r[...] = x
        xw_r[...] = jnp.dot(x, w1t_r[...],
                            preferred_element_type=jnp.float32) + b1_r[...]

    vec = pl.BlockSpec((1, H), lambda i: (0, 0))
    mat = pl.BlockSpec((H, H), lambda i: (0, 0))
    blk = pl.BlockSpec((NBL, H), lambda i: (i, 0))
    return pl.pallas_call(
        body,
        grid=(NP // NBL,),
        in_specs=[blk, mat, vec, vec, vec, mat, vec],
        out_specs=[blk, blk],
        out_shape=[jax.ShapeDtypeStruct((NP, H), jnp.float32)] * 2,
        compiler_params=pltpu.CompilerParams(
            dimension_semantics=("arbitrary",)),
    )(nf, nW, nb, ng, nbe, w1t, b1)


def _tc_messages(ef, gx, eW, eb, eg, ebe, w1b, g1, be1, w2, b2, g2, be2):
    def body(ef_r, gx_r, eW_r, eb_r, eg_r, ebe_r, w1b_r, g1_r, be1_r,
             w2_r, b2_r, g2_r, be2_r, m_r):
        ea = jnp.dot(ef_r[...], eW_r[...], preferred_element_type=jnp.float32)
        ea = _gelu_in(_ln_in(ea + eb_r[...], eg_r[...], ebe_r[...]))
        pre = gx_r[...] + jnp.dot(ea, w1b_r[...],
                                  preferred_element_type=jnp.float32)
        h = _gelu_in(_ln_in(pre, g1_r[...], be1_r[...]))
        m = jnp.dot(h, w2_r[...], preferred_element_type=jnp.float32)
        m_r[...] = _gelu_in(_ln_in(m + b2_r[...], g2_r[...], be2_r[...]))

    vec = pl.BlockSpec((1, H), lambda i: (0, 0))
    mat = pl.BlockSpec((H, H), lambda i: (0, 0))
    return pl.pallas_call(
        body,
        grid=(ECH // EB,),
        in_specs=[
            pl.BlockSpec((EB, DE), lambda i: (i, 0)),
            pl.BlockSpec((EB, H), lambda i: (i, 0)),
            pl.BlockSpec((DE, H), lambda i: (0, 0)),
            vec, vec, vec, mat, vec, vec, mat, vec, vec, vec,
        ],
        out_specs=pl.BlockSpec((EB, H), lambda i: (i, 0)),
        out_shape=jax.ShapeDtypeStruct((ECH, H), jnp.float32),
        compiler_params=pltpu.CompilerParams(
            dimension_semantics=("arbitrary",)),
    )(ef, gx, eW, eb, eg, ebe, w1b, g1, be1, w2, b2, g2, be2)


def _agg_from_parts(acc_refs, c_r):
    cnt = c_r[0, :, 0:1] + c_r[1, :, 0:1]
    s = acc_refs[0][0] + acc_refs[0][1]
    for a_r in acc_refs[1:]:
        s = s + a_r[0] + a_r[1]
    return s / jnp.maximum(cnt, 1.0)


def _tc_update(x, parts, cnt2, w1t, b1):
    npart = len(parts)

    def body(*refs):
        x_r = refs[0]
        accs = refs[1:1 + npart]
        c_r, w1t_r, b1_r, xn_r, xw_r = refs[1 + npart:]
        xn = x_r[...] + _agg_from_parts(accs, c_r)
        xn_r[...] = xn
        xw_r[...] = jnp.dot(xn, w1t_r[...],
                            preferred_element_type=jnp.float32) + b1_r[...]

    vec = pl.BlockSpec((1, H), lambda i: (0, 0))
    mat = pl.BlockSpec((H, H), lambda i: (0, 0))
    blk = pl.BlockSpec((NBL, H), lambda i: (i, 0))
    acc = pl.BlockSpec((2, NBL, H), lambda i: (0, i, 0))
    return pl.pallas_call(
        body,
        grid=(NP // NBL,),
        in_specs=[blk] + [acc] * npart + [acc, mat, vec],
        out_specs=[blk, blk],
        out_shape=[jax.ShapeDtypeStruct((NP, H), jnp.float32)] * 2,
        compiler_params=pltpu.CompilerParams(
            dimension_semantics=("arbitrary",)),
    )(x, *parts, cnt2, w1t, b1)


def _tc_final(x, parts, cnt2, bi, gnorm, bnorm):
    nblk = NP // NBL
    npart = len(parts)

    def body(*refs):
        x_r = refs[0]
        accs = refs[1:1 + npart]
        c_r, bi_r, g_r, b_r, out_r, gsum, gcnt = refs[1 + npart:]
        i = pl.program_id(0)

        @pl.when(i == 0)
        def _():
            gsum[...] = jnp.zeros_like(gsum)
            gcnt[...] = jnp.zeros_like(gcnt)

        xn = x_r[...] + _agg_from_parts(accs, c_r)
        xn = _ln_in(xn, g_r[...], b_r[...])
        bib = jnp.broadcast_to(bi_r[...], (NG, NBL))
        iot = lax.broadcasted_iota(jnp.int32, (NG, NBL), 0).astype(jnp.float32)
        oh = jnp.where(iot == bib, 1.0, 0.0)
        gsum[...] += jnp.dot(oh, xn, preferred_element_type=jnp.float32)
        gcnt[...] += jnp.broadcast_to(
            jnp.sum(oh, axis=1, keepdims=True), (NG, H))

        @pl.when(i == nblk - 1)
        def _():
            out_r[...] = gsum[...] / jnp.maximum(gcnt[...], 1.0)

    vec = pl.BlockSpec((1, H), lambda i: (0, 0))
    acc = pl.BlockSpec((2, NBL, H), lambda i: (0, i, 0))
    return pl.pallas_call(
        body,
        grid=(nblk,),
        in_specs=[pl.BlockSpec((NBL, H), lambda i: (i, 0))]
        + [acc] * npart
        + [acc, pl.BlockSpec((1, NBL), lambda i: (0, i)), vec, vec],
        out_specs=pl.BlockSpec((NG, H), lambda i: (0, 0)),
        out_shape=jax.ShapeDtypeStruct((NG, H), jnp.float32),
        scratch_shapes=[pltpu.VMEM((NG, H), jnp.float32)] * 2,
        compiler_params=pltpu.CompilerParams(
            dimension_semantics=("arbitrary",)),
    )(x, *parts, cnt2, bi, gnorm, bnorm)


def kernel(node_feats, edge_feats, edge_index, batch_index, params):
    f32 = jnp.float32
    p = params
    layers = p['layers']

    row4 = edge_index[0].astype(jnp.int32).reshape(NCHUNK, NW, NCHC, CH)
    col_i32 = edge_index[1].astype(jnp.int32)
    col3 = col_i32.reshape(NW, NCH, CH)
    col4s = col_i32.reshape(NCHUNK, NW, NCHSC, CHS)
    ef4 = edge_feats.reshape(NCHUNK, ECH, DE)
    nf_p = jnp.pad(node_feats, ((0, NP - NN), (0, 0)))
    bi = jnp.pad(batch_index.astype(f32), (0, NP - NN),
                 constant_values=float(NG)).reshape(1, NP)
    z128 = jnp.zeros((RPT, H), f32)
    ones128 = jnp.ones((CH, H), f32)

    def v2d(v):
        return v.reshape(1, H)

    w1t = [lp['W1'][:H] for lp in layers]
    w1b = [lp['W1'][H:] for lp in layers]

    cnt2 = _sc_counts(col3, z128, ones128)

    x, xw = _tc_node_encoder(nf_p, p['node_W'], v2d(p['node_b']),
                             v2d(p['node_g']), v2d(p['node_be']),
                             w1t[0], v2d(layers[0]['b1']))

    parts = None
    for i in range(3):
        lp = layers[i]
        parts = []
        for kc in range(NCHUNK):
            gx = _sc_gather(xw, row4[kc])
            m = _tc_messages(ef4[kc], gx, p['edge_W'], v2d(p['edge_b']),
                             v2d(p['edge_g']), v2d(p['edge_be']), w1b[i],
                             v2d(lp['g1']), v2d(lp['be1']), lp['W2'],
                             v2d(lp['b2']), v2d(lp['g2']), v2d(lp['be2']))
            parts.append(_sc_scatter(m, col4s[kc], z128))
        if i < 2:
            x, xw = _tc_update(x, parts, cnt2, w1t[i + 1],
                               v2d(layers[i + 1]['b1']))

    return _tc_final(x, parts, cnt2, bi, v2d(p['norm_g']), v2d(p['norm_b']))


# speedup vs baseline: 1.0563x; 1.0188x over previous
import functools

import jax
import jax.numpy as jnp
from jax import lax
from jax.experimental import pallas as pl
from jax.experimental.pallas import tpu as pltpu
from jax.experimental.pallas import tpu_sc as plsc

NN = 10000
NP = 10240
E = 320000
DE = 16
H = 128
NG = 16

NW = 32

NCHUNK = 5
ECH = E // NCHUNK
EPWC = ECH // NW

CH = 80
NCHC = EPWC // CH
NB = 5
NGRPC = NCHC // NB

CHS = 40
NCHSC = EPWC // CHS
NBS = 5
NGRPSC = NCHSC // NBS

EPW = E // NW
NCH = EPW // CH
RPT = NP // 16

NBL = 1024
EB = 2000

_MESH_KW = dict(core_axis_name="c", subcore_axis_name="s")


def _sc_gather(xw, rowc):
    mesh = plsc.VectorSubcoreMesh(**_MESH_KW)

    @functools.partial(
        pl.kernel,
        out_type=jax.ShapeDtypeStruct((ECH, H), jnp.float32),
        mesh=mesh,
        scratch_types=[
            pltpu.VMEM((NCHC, CH), jnp.int32),
            pltpu.VMEM((NB, CH, H), jnp.float32),
            pltpu.SemaphoreType.DMA,
            pltpu.SemaphoreType.DMA,
        ],
    )
    def k(xw_hbm, row_hbm, out_hbm, idx_v, buf_v, gsem, ssem):
        wid = lax.axis_index("c") * 16 + lax.axis_index("s")
        base = wid * EPWC
        pltpu.sync_copy(row_hbm.at[wid], idx_v)

        @pl.loop(0, NGRPC)
        def _grp(g):
            j0 = g * NB
            for b in range(NB):
                pltpu.async_copy(xw_hbm.at[idx_v.at[j0 + b]], buf_v.at[b], gsem)
            for b in range(NB):
                pltpu.make_async_copy(
                    xw_hbm.at[idx_v.at[j0 + b]], buf_v.at[b], gsem).wait()
            for b in range(NB):
                pltpu.async_copy(
                    buf_v.at[b], out_hbm.at[pl.ds(base + (j0 + b) * CH, CH)], ssem)
            for b in range(NB):
                pltpu.make_async_copy(
                    buf_v.at[b], out_hbm.at[pl.ds(base + (j0 + b) * CH, CH)], ssem).wait()

    return k(xw, rowc)


def _sc_scatter(m, colcs, zrows):
    mesh = plsc.VectorSubcoreMesh(**_MESH_KW)

    @functools.partial(
        pl.kernel,
        out_type=jax.ShapeDtypeStruct((2, NP, H), jnp.float32),
        mesh=mesh,
        scratch_types=[
            pltpu.VMEM((NCHSC, CHS), jnp.int32),
            pltpu.VMEM((NBS, CHS, H), jnp.float32),
            pltpu.VMEM_SHARED((NP, H), jnp.float32),
            pltpu.SemaphoreType.DMA,
        ],
    )
    def k(m_hbm, col_hbm, z_hbm, out_hbm, idx_v, buf_v, acc_sh, lsem):
        cid = lax.axis_index("c")
        sid = lax.axis_index("s")
        wid = cid * 16 + sid
        base = wid * EPWC
        pltpu.sync_copy(z_hbm, acc_sh.at[pl.ds(sid * RPT, RPT)])
        pltpu.sync_copy(col_hbm.at[wid], idx_v)
        plsc.subcore_barrier()

        def _issue(g):
            for b in range(NBS):
                pltpu.async_copy(
                    m_hbm.at[pl.ds(base + (g * NBS + b) * CHS, CHS)],
                    buf_v.at[b], lsem)

        _issue(0)

        @pl.loop(0, NGRPSC)
        def _grp(g):
            for b in range(NBS):
                pltpu.make_async_copy(
                    m_hbm.at[pl.ds(base + (g * NBS + b) * CHS, CHS)],
                    buf_v.at[b], lsem).wait()
            for b in range(NBS):
                pltpu.sync_copy(buf_v.at[b], acc_sh.at[idx_v.at[g * NBS + b]], add=True)

            @pl.when(g + 1 < NGRPSC)
            def _():
                _issue(g + 1)

        plsc.subcore_barrier()
        pltpu.sync_copy(acc_sh.at[pl.ds(sid * RPT, RPT)],
                        out_hbm.at[cid, pl.ds(sid * RPT, RPT)])

    return k(m, colcs, zrows)


def _sc_counts(col3, zrows, ones_rows):
    mesh = plsc.VectorSubcoreMesh(**_MESH_KW)

    @functools.partial(
        pl.kernel,
        out_type=jax.ShapeDtypeStruct((2, NP, H), jnp.float32),
        mesh=mesh,
        scratch_types=[
            pltpu.VMEM((NCH, CH), jnp.int32),
            pltpu.VMEM((CH, H), jnp.float32),
            pltpu.VMEM_SHARED((NP, H), jnp.float32),
        ],
    )
    def k(col_hbm, z_hbm, ones_hbm, out_hbm, idx_v, ones_v, acc_sh):
        cid = lax.axis_index("c")
        sid = lax.axis_index("s")
        wid = cid * 16 + sid
        pltpu.sync_copy(z_hbm, acc_sh.at[pl.ds(sid * RPT, RPT)])
        pltpu.sync_copy(ones_hbm, ones_v)
        pltpu.sync_copy(col_hbm.at[wid], idx_v)
        plsc.subcore_barrier()

        @pl.loop(0, NCH)
        def _j(j):
            pltpu.sync_copy(ones_v, acc_sh.at[idx_v.at[j]], add=True)

        plsc.subcore_barrier()
        pltpu.sync_copy(acc_sh.at[pl.ds(sid * RPT, RPT)],
                        out_hbm.at[cid, pl.ds(sid * RPT, RPT)])

    return k(col3, zrows, ones_rows)


def _ln_in(x, g, b):
    mu = jnp.mean(x, axis=-1, keepdims=True)
    var = jnp.mean((x - mu) ** 2, axis=-1, keepdims=True)
    return (x - mu) * lax.rsqrt(var + 1e-5) * g + b


def _gelu_in(x):
    return 0.5 * x * (1.0 + lax.erf(x * 0.7071067811865476))


def _tc_node_encoder(nf, nW, nb, ng, nbe, w1t, b1):
    def body(nf_r, nW_r, nb_r, ng_r, nbe_r, w1t_r, b1_r, x_r, xw_r):
        t = jnp.dot(nf_r[...], nW_r[...], preferred_element_type=jnp.float32)
        x = _gelu_in(_ln_in(t + nb_r[...], ng_r[...], nbe_r[...]))
        x_r[...] = x
        xw_r[...] = jnp.dot(x, w1t_r[...],
                            preferred_element_type=jnp.float32) + b1_r[...]

    vec = pl.BlockSpec((1, H), lambda i: (0, 0))
    mat = pl.BlockSpec((H, H), lambda i: (0, 0))
    blk = pl.BlockSpec((NBL, H), lambda i: (i, 0))
    return pl.pallas_call(
        body,
        grid=(NP // NBL,),
        in_specs=[blk, mat, vec, vec, vec, mat, vec],
        out_specs=[blk, blk],
        out_shape=[jax.ShapeDtypeStruct((NP, H), jnp.float32)] * 2,
        compiler_params=pltpu.CompilerParams(
            dimension_semantics=("arbitrary",)),
    )(nf, nW, nb, ng, nbe, w1t, b1)


def _tc_messages(ef, gx, eW, eb, eg, ebe, w1b, g1, be1, w2, b2, g2, be2):
    def body(ef_r, gx_r, eW_r, eb_r, eg_r, ebe_r, w1b_r, g1_r, be1_r,
             w2_r, b2_r, g2_r, be2_r, m_r):
        ea = jnp.dot(ef_r[...], eW_r[...], preferred_element_type=jnp.float32)
        ea = _gelu_in(_ln_in(ea + eb_r[...], eg_r[...], ebe_r[...]))
        pre = gx_r[...] + jnp.dot(ea, w1b_r[...],
                                  preferred_element_type=jnp.float32)
        h = _gelu_in(_ln_in(pre, g1_r[...], be1_r[...]))
        m = jnp.dot(h, w2_r[...], preferred_element_type=jnp.float32)
        m_r[...] = _gelu_in(_ln_in(m + b2_r[...], g2_r[...], be2_r[...]))

    vec = pl.BlockSpec((1, H), lambda i: (0, 0))
    mat = pl.BlockSpec((H, H), lambda i: (0, 0))
    return pl.pallas_call(
        body,
        grid=(ECH // EB,),
        in_specs=[
            pl.BlockSpec((EB, DE), lambda i: (i, 0)),
            pl.BlockSpec((EB, H), lambda i: (i, 0)),
            pl.BlockSpec((DE, H), lambda i: (0, 0)),
            vec, vec, vec, mat, vec, vec, mat, vec, vec, vec,
        ],
        out_specs=pl.BlockSpec((EB, H), lambda i: (i, 0)),
        out_shape=jax.ShapeDtypeStruct((ECH, H), jnp.float32),
        compiler_params=pltpu.CompilerParams(
            dimension_semantics=("arbitrary",)),
    )(ef, gx, eW, eb, eg, ebe, w1b, g1, be1, w2, b2, g2, be2)


def _agg_from_parts(acc_refs, c_r):
    cnt = c_r[0, :, 0:1] + c_r[1, :, 0:1]
    s = acc_refs[0][0] + acc_refs[0][1]
    for a_r in acc_refs[1:]:
        s = s + a_r[0] + a_r[1]
    return s / jnp.maximum(cnt, 1.0)


def _tc_update(x, parts, cnt2, w1t, b1):
    npart = len(parts)

    def body(*refs):
        x_r = refs[0]
        accs = refs[1:1 + npart]
        c_r, w1t_r, b1_r, xn_r, xw_r = refs[1 + npart:]
        xn = x_r[...] + _agg_from_parts(accs, c_r)
        xn_r[...] = xn
        xw_r[...] = jnp.dot(xn, w1t_r[...],
                            preferred_element_type=jnp.float32) + b1_r[...]

    vec = pl.BlockSpec((1, H), lambda i: (0, 0))
    mat = pl.BlockSpec((H, H), lambda i: (0, 0))
    blk = pl.BlockSpec((NBL, H), lambda i: (i, 0))
    acc = pl.BlockSpec((2, NBL, H), lambda i: (0, i, 0))
    return pl.pallas_call(
        body,
        grid=(NP // NBL,),
        in_specs=[blk] + [acc] * npart + [acc, mat, vec],
        out_specs=[blk, blk],
        out_shape=[jax.ShapeDtypeStruct((NP, H), jnp.float32)] * 2,
        compiler_params=pltpu.CompilerParams(
            dimension_semantics=("arbitrary",)),
    )(x, *parts, cnt2, w1t, b1)


def _tc_final(x, parts, cnt2, bi, gnorm, bnorm):
    nblk = NP // NBL
    npart = len(parts)

    def body(*refs):
        x_r = refs[0]
        accs = refs[1:1 + npart]
        c_r, bi_r, g_r, b_r, out_r, gsum, gcnt = refs[1 + npart:]
        i = pl.program_id(0)

        @pl.when(i == 0)
        def _():
            gsum[...] = jnp.zeros_like(gsum)
            gcnt[...] = jnp.zeros_like(gcnt)

        xn = x_r[...] + _agg_from_parts(accs, c_r)
        xn = _ln_in(xn, g_r[...], b_r[...])
        bib = jnp.broadcast_to(bi_r[...], (NG, NBL))
        iot = lax.broadcasted_iota(jnp.int32, (NG, NBL), 0).astype(jnp.float32)
        oh = jnp.where(iot == bib, 1.0, 0.0)
        gsum[...] += jnp.dot(oh, xn, preferred_element_type=jnp.float32)
        gcnt[...] += jnp.broadcast_to(
            jnp.sum(oh, axis=1, keepdims=True), (NG, H))

        @pl.when(i == nblk - 1)
        def _():
            out_r[...] = gsum[...] / jnp.maximum(gcnt[...], 1.0)

    vec = pl.BlockSpec((1, H), lambda i: (0, 0))
    acc = pl.BlockSpec((2, NBL, H), lambda i: (0, i, 0))
    return pl.pallas_call(
        body,
        grid=(nblk,),
        in_specs=[pl.BlockSpec((NBL, H), lambda i: (i, 0))]
        + [acc] * npart
        + [acc, pl.BlockSpec((1, NBL), lambda i: (0, i)), vec, vec],
        out_specs=pl.BlockSpec((NG, H), lambda i: (0, 0)),
        out_shape=jax.ShapeDtypeStruct((NG, H), jnp.float32),
        scratch_shapes=[pltpu.VMEM((NG, H), jnp.float32)] * 2,
        compiler_params=pltpu.CompilerParams(
            dimension_semantics=("arbitrary",)),
    )(x, *parts, cnt2, bi, gnorm, bnorm)


def kernel(node_feats, edge_feats, edge_index, batch_index, params):
    f32 = jnp.float32
    p = params
    layers = p['layers']

    row4 = edge_index[0].astype(jnp.int32).reshape(NCHUNK, NW, NCHC, CH)
    col_i32 = edge_index[1].astype(jnp.int32)
    col3 = col_i32.reshape(NW, NCH, CH)
    col4s = col_i32.reshape(NCHUNK, NW, NCHSC, CHS)
    ef4 = edge_feats.reshape(NCHUNK, ECH, DE)
    nf_p = jnp.pad(node_feats, ((0, NP - NN), (0, 0)))
    bi = jnp.pad(batch_index.astype(f32), (0, NP - NN),
                 constant_values=float(NG)).reshape(1, NP)
    z128 = jnp.zeros((RPT, H), f32)
    ones128 = jnp.ones((CH, H), f32)

    def v2d(v):
        return v.reshape(1, H)

    w1t = [lp['W1'][:H] for lp in layers]
    w1b = [lp['W1'][H:] for lp in layers]

    cnt2 = _sc_counts(col3, z128, ones128)

    x, xw = _tc_node_encoder(nf_p, p['node_W'], v2d(p['node_b']),
                             v2d(p['node_g']), v2d(p['node_be']),
                             w1t[0], v2d(layers[0]['b1']))

    parts = None
    for i in range(3):
        lp = layers[i]
        parts = []
        for kc in range(NCHUNK):
            gx = _sc_gather(xw, row4[kc])
            m = _tc_messages(ef4[kc], gx, p['edge_W'], v2d(p['edge_b']),
                             v2d(p['edge_g']), v2d(p['edge_be']), w1b[i],
                             v2d(lp['g1']), v2d(lp['be1']), lp['W2'],
                             v2d(lp['b2']), v2d(lp['g2']), v2d(lp['be2']))
            parts.append(_sc_scatter(m, col4s[kc], z128))
        if i < 2:
            x, xw = _tc_update(x, parts, cnt2, w1t[i + 1],
                               v2d(layers[i + 1]['b1']))

    return _tc_final(x, parts, cnt2, bi, v2d(p['norm_g']), v2d(p['norm_b']))
